# Initial kernel scaffold; baseline (speedup 1.0000x reference)
import functools

import jax
import jax.numpy as jnp
from jax import lax
from jax.experimental import pallas as pl
from jax.experimental.pallas import tpu as pltpu
from jax.experimental.pallas import tpu_sc as plsc

NEG_SLOPE = 0.2
LAM = 0.5
N_HOPS = 2
NW = 32
LANES = 16


def _rup(x, m):
    return (x + m - 1) // m * m


def _proj_body(ent_ref, w_ref, rel_ref, p1_ref, p2_ref):
    d = ent_ref.shape[1]
    cdims = (((1,), (1,)), ((), ()))
    m1 = lax.dot_general(w_ref[0:d, :], rel_ref[...], cdims,
                         preferred_element_type=jnp.float32)
    m2 = lax.dot_general(w_ref[d:2 * d, :], rel_ref[...], cdims,
                         preferred_element_type=jnp.float32)
    e = ent_ref[...]
    p1_ref[...] = jnp.dot(e, m1, preferred_element_type=jnp.float32)
    p2_ref[...] = jnp.dot(e, m2, preferred_element_type=jnp.float32)


@functools.lru_cache(maxsize=None)
def _make_proj(n, d, r, blk):
    grid = n // blk
    return pl.pallas_call(
        _proj_body,
        grid=(grid,),
        in_specs=[
            pl.BlockSpec((blk, d), lambda i: (i, 0)),
            pl.BlockSpec((2 * d, d), lambda i: (0, 0)),
            pl.BlockSpec((r, d), lambda i: (0, 0)),
        ],
        out_specs=[
            pl.BlockSpec((blk, r), lambda i: (i, 0)),
            pl.BlockSpec((blk, r), lambda i: (i, 0)),
        ],
        out_shape=[
            jax.ShapeDtypeStruct((n, r), jnp.float32),
            jax.ShapeDtypeStruct((n, r), jnp.float32),
        ],
    )


def _norm_body(a0_ref, a1_ref, ent_ref, res_ref, oent_ref, ores_ref):
    a = a0_ref[...] + a1_ref[...] + ent_ref[...]
    nrm = jnp.sqrt(jnp.sum(a * a, axis=1, keepdims=True))
    ent_new = a / jnp.maximum(nrm, 1e-12)
    oent_ref[...] = ent_new
    ores_ref[...] = LAM * res_ref[...] + ent_new


@functools.lru_cache(maxsize=None)
def _make_norm(n, d, blk):
    grid = n // blk
    return pl.pallas_call(
        _norm_body,
        grid=(grid,),
        in_specs=[
            pl.BlockSpec((blk, d), lambda i: (i, 0)),
            pl.BlockSpec((blk, d), lambda i: (i, 0)),
            pl.BlockSpec((blk, d), lambda i: (i, 0)),
            pl.BlockSpec((blk, d), lambda i: (i, 0)),
        ],
        out_specs=[
            pl.BlockSpec((blk, d), lambda i: (i, 0)),
            pl.BlockSpec((blk, d), lambda i: (i, 0)),
        ],
        out_shape=[
            jax.ShapeDtypeStruct((n, d), jnp.float32),
            jax.ShapeDtypeStruct((n, d), jnp.float32),
        ],
    )


def _mesh():
    return plsc.VectorSubcoreMesh(core_axis_name="c", subcore_axis_name="s")


def _wid():
    return lax.axis_index("s") * 2 + lax.axis_index("c")


@functools.lru_cache(maxsize=None)
def _make_sc_logits(rows_w, r):
    nrows = NW * rows_w

    @functools.partial(
        pl.kernel, mesh=_mesh(),
        out_type=(jax.ShapeDtypeStruct((nrows, 128), jnp.float32),
                  jax.ShapeDtypeStruct((NW, LANES), jnp.float32)),
        scratch_types=[
            pltpu.VMEM((rows_w, 128), jnp.int32),
            pltpu.VMEM((rows_w, 128), jnp.int32),
            pltpu.VMEM((rows_w, 128), jnp.int32),
            pltpu.VMEM((rows_w, 128), jnp.int32),
            pltpu.VMEM((rows_w, 128), jnp.int32),
            pltpu.VMEM((rows_w, 128), jnp.float32),
            pltpu.VMEM((rows_w, 128), jnp.float32),
            pltpu.VMEM((rows_w, 128), jnp.float32),
            pltpu.VMEM((LANES,), jnp.float32),
            pltpu.SemaphoreType.DMA,
            pltpu.SemaphoreType.DMA,
        ])
    def k(p1f, p2f, head2, tail2, type2, e_out, pmax_out,
          hbuf, tbuf, ybuf, i1, i2, v1, v2, ebuf, mbuf, sem1, sem2):
        w = _wid()
        base = w * rows_w
        pltpu.sync_copy(head2.at[pl.ds(base, rows_w)], hbuf)
        pltpu.sync_copy(tail2.at[pl.ds(base, rows_w)], tbuf)
        pltpu.sync_copy(type2.at[pl.ds(base, rows_w)], ybuf)

        def idx_row(j, c):
            for kk in range(8):
                sl = pl.ds(kk * LANES, LANES)
                y = ybuf[j, sl]
                i1[j, sl] = hbuf[j, sl] * r + y
                i2[j, sl] = tbuf[j, sl] * r + y
            return c
        lax.fori_loop(0, rows_w, idx_row, 0)

        cp1 = pltpu.async_copy(p1f.at[i1], v1, sem1)
        cp2 = pltpu.async_copy(p2f.at[i2], v2, sem2)
        cp1.wait()
        cp2.wait()

        def e_row(j, m):
            for kk in range(8):
                sl = pl.ds(kk * LANES, LANES)
                s = v1[j, sl] + v2[j, sl]
                ev = jnp.where(s >= 0.0, s, NEG_SLOPE * s)
                ebuf[j, sl] = ev
                m = jnp.maximum(m, ev)
            return m
        m = lax.fori_loop(0, rows_w, e_row,
                          jnp.full((LANES,), -3e38, jnp.float32))
        mbuf[...] = m
        pltpu.sync_copy(ebuf, e_out.at[pl.ds(base, rows_w)])
        pltpu.sync_copy(mbuf, pmax_out.at[w])

    return k


@functools.lru_cache(maxsize=None)
def _make_sc_denom(rows_w, e_real, npad):
    nrows = NW * rows_w
    stripe = npad // LANES

    @functools.partial(
        pl.kernel, mesh=_mesh(),
        out_type=(jax.ShapeDtypeStruct((nrows, 128), jnp.float32),
                  jax.ShapeDtypeStruct((2, npad), jnp.float32)),
        scratch_types=[
            pltpu.VMEM((rows_w, 128), jnp.float32),
            pltpu.VMEM((rows_w, 128), jnp.float32),
            pltpu.VMEM((rows_w, 128), jnp.int32),
            pltpu.VMEM((NW, LANES), jnp.float32),
            pltpu.VMEM((npad // LANES,), jnp.float32),
            pltpu.VMEM_SHARED((npad,), jnp.float32),
        ])
    def k(e2, pmax, head2, ex_out, den_out, ebuf, xbuf, hbuf, pv, zbuf, dsh):
        w = _wid()
        cid = lax.axis_index("c")
        sid = lax.axis_index("s")
        base = w * rows_w
        pltpu.sync_copy(e2.at[pl.ds(base, rows_w)], ebuf)
        pltpu.sync_copy(head2.at[pl.ds(base, rows_w)], hbuf)
        pltpu.sync_copy(pmax, pv)

        def mrow(i, m):
            return jnp.maximum(m, pv[i])
        gmax = jnp.max(lax.fori_loop(
            0, NW, mrow, jnp.full((LANES,), -3e38, jnp.float32)))

        stripe_n = npad // LANES

        def zrow(q, c):
            zbuf[pl.ds(q * LANES, LANES)] = jnp.zeros((LANES,), jnp.float32)
            return c
        lax.fori_loop(0, stripe_n // LANES, zrow, 0)
        pltpu.sync_copy(zbuf, dsh.at[pl.ds(sid * stripe_n, stripe_n)])
        plsc.subcore_barrier()

        iota = lax.iota(jnp.int32, LANES)

        def x_row(j, c):
            gid0 = (base + j) * 128
            for kk in range(8):
                sl = pl.ds(kk * LANES, LANES)
                x = jnp.exp(ebuf[j, sl] - gmax)
                gid = gid0 + kk * LANES + iota
                xbuf[j, sl] = jnp.where(gid < e_real, x, 0.0)
            return c
        lax.fori_loop(0, rows_w, x_row, 0)

        pltpu.sync_copy(xbuf, ex_out.at[pl.ds(base, rows_w)])
        pltpu.sync_copy(xbuf, dsh.at[hbuf], add=True)
        plsc.subcore_barrier()
        pltpu.sync_copy(dsh.at[pl.ds(sid * stripe_n, stripe_n)],
                        den_out.at[cid, pl.ds(sid * stripe_n, stripe_n)])

    return k


@functools.lru_cache(maxsize=None)
def _make_sc_agg(rows_w, npad, d):
    stripe = npad // LANES

    @functools.partial(
        pl.kernel, mesh=_mesh(),
        out_type=jax.ShapeDtypeStruct((2, npad, d), jnp.float32),
        scratch_types=[
            pltpu.VMEM((rows_w, 128), jnp.int32),
            pltpu.VMEM((rows_w, 128), jnp.int32),
            pltpu.VMEM((rows_w, 128), jnp.float32),
            pltpu.VMEM((rows_w, 128), jnp.float32),
            pltpu.VMEM((npad,), jnp.float32),
            pltpu.VMEM((npad,), jnp.float32),
            pltpu.VMEM((128, d), jnp.float32),
            pltpu.VMEM_SHARED((npad, d), jnp.float32),
            pltpu.SemaphoreType.DMA,
        ])
    def k(ex2, den2, head2, tail2, ent, agg_out,
          hbuf, tbuf, xbuf, abuf, d0, d1, rows, ash, sem):
        w = _wid()
        cid = lax.axis_index("c")
        sid = lax.axis_index("s")
        base = w * rows_w
        pltpu.sync_copy(head2.at[pl.ds(base, rows_w)], hbuf)
        pltpu.sync_copy(tail2.at[pl.ds(base, rows_w)], tbuf)
        pltpu.sync_copy(ex2.at[pl.ds(base, rows_w)], xbuf)
        pltpu.sync_copy(den2.at[0], d0)
        pltpu.sync_copy(den2.at[1], d1)

        def dsum(q, c):
            sl = pl.ds(q * LANES, LANES)
            d1[sl] = jnp.maximum(d0[sl] + d1[sl], 1e-30)
            return c
        lax.fori_loop(0, npad // LANES, dsum, 0)

        def a_row(j, c):
            for kk in range(8):
                sl = pl.ds(kk * LANES, LANES)
                dn = plsc.load_gather(d1, [hbuf[j, sl]])
                abuf[j, sl] = xbuf[j, sl] / dn
            return c
        lax.fori_loop(0, rows_w, a_row, 0)

        def zr(i, c):
            for kk in range(d // LANES):
                rows[i, pl.ds(kk * LANES, LANES)] = jnp.zeros((LANES,),
                                                              jnp.float32)
            return c
        lax.fori_loop(0, 128, zr, 0)

        def zcopy(q, c):
            pltpu.sync_copy(rows, ash.at[pl.ds(sid * stripe + q * 128, 128)])
            return c
        lax.fori_loop(0, stripe // 128, zcopy, 0)
        plsc.subcore_barrier()

        def chunk(j, c):
            pltpu.async_copy(ent.at[tbuf.at[j]], rows, sem).wait()

            def srow(i, cc):
                a = abuf[j, i]
                for kk in range(d // LANES):
                    sl = pl.ds(kk * LANES, LANES)
                    rows[i, sl] = rows[i, sl] * a
                return cc
            lax.fori_loop(0, 128, srow, 0)
            pltpu.sync_copy(rows, ash.at[hbuf.at[j]], add=True)
            return c
        lax.fori_loop(0, rows_w, chunk, 0)
        plsc.subcore_barrier()

        pltpu.sync_copy(ash.at[pl.ds(sid * stripe, stripe)],
                        agg_out.at[cid, pl.ds(sid * stripe, stripe)])

    return k


def kernel(entity_emb, relation_emb, edge_index, edge_type, W):
    n, d = entity_emb.shape
    r = relation_emb.shape[0]
    e_real = edge_type.shape[0]
    rows_w = -(-e_real // (NW * 128))
    epad = NW * rows_w * 128
    npad = _rup(n, LANES * 128)
    blk = 1000 if n % 1000 == 0 else 8

    head = edge_index[0].astype(jnp.int32)
    tail = edge_index[1].astype(jnp.int32)
    etype = edge_type.astype(jnp.int32)
    pad = epad - e_real
    zpad = jnp.zeros((pad,), jnp.int32)
    head2 = jnp.concatenate([head, zpad]).reshape(NW * rows_w, 128)
    tail2 = jnp.concatenate([tail, zpad]).reshape(NW * rows_w, 128)
    type2 = jnp.concatenate([etype, zpad]).reshape(NW * rows_w, 128)

    proj = _make_proj(n, d, r, blk)
    norm = _make_norm(n, d, blk)
    sc_a = _make_sc_logits(rows_w, r)
    sc_b = _make_sc_denom(rows_w, e_real, npad)
    sc_c = _make_sc_agg(rows_w, npad, d)

    ent = entity_emb
    res = entity_emb
    for _ in range(N_HOPS):
        p1, p2 = proj(ent, W, relation_emb)
        p1f = p1.reshape(n * r)
        p2f = p2.reshape(n * r)
        e2, pmax = sc_a(p1f, p2f, head2, tail2, type2)
        ex2, den2 = sc_b(e2, pmax, head2)
        agg = sc_c(ex2, den2, head2, tail2, ent)
        ent, res = norm(agg[0, :n], agg[1, :n], ent, res)
    return res


# speedup vs baseline: 8.3264x; 8.3264x over previous
import functools

import jax
import jax.numpy as jnp
from jax import lax
from jax.experimental import pallas as pl
from jax.experimental.pallas import tpu as pltpu
from jax.experimental.pallas import tpu_sc as plsc

NEG_SLOPE = 0.2
LAM = 0.5
N_HOPS = 2
NW = 32
LANES = 16


def _rup(x, m):
    return (x + m - 1) // m * m


def _proj_body(ent_ref, w_ref, rel_ref, p1_ref, p2_ref):
    d = ent_ref.shape[1]
    cdims = (((1,), (1,)), ((), ()))
    m1 = lax.dot_general(w_ref[0:d, :], rel_ref[...], cdims,
                         preferred_element_type=jnp.float32)
    m2 = lax.dot_general(w_ref[d:2 * d, :], rel_ref[...], cdims,
                         preferred_element_type=jnp.float32)
    e = ent_ref[...]
    p1_ref[...] = jnp.dot(e, m1, preferred_element_type=jnp.float32)
    p2_ref[...] = jnp.dot(e, m2, preferred_element_type=jnp.float32)


@functools.lru_cache(maxsize=None)
def _make_proj(n, d, r, blk):
    grid = n // blk
    return pl.pallas_call(
        _proj_body,
        grid=(grid,),
        in_specs=[
            pl.BlockSpec((blk, d), lambda i: (i, 0)),
            pl.BlockSpec((2 * d, d), lambda i: (0, 0)),
            pl.BlockSpec((r, d), lambda i: (0, 0)),
        ],
        out_specs=[
            pl.BlockSpec((blk, r), lambda i: (i, 0)),
            pl.BlockSpec((blk, r), lambda i: (i, 0)),
        ],
        out_shape=[
            jax.ShapeDtypeStruct((n, r), jnp.float32),
            jax.ShapeDtypeStruct((n, r), jnp.float32),
        ],
    )


def _shift_body(pmax_ref, den_ref, s_ref):
    gmax = jnp.max(pmax_ref[...])
    dt = den_ref[0] + den_ref[1]
    dts = jnp.where(dt > 0.0, dt, 1.0)
    s_ref[...] = jnp.where(dt > 0.0, gmax + jnp.log(dts), gmax - 88.0)


@functools.lru_cache(maxsize=None)
def _make_shift(npad):
    rows = npad // 128
    return pl.pallas_call(
        _shift_body,
        grid=(1,),
        in_specs=[
            pl.BlockSpec((NW, LANES), lambda i: (0, 0)),
            pl.BlockSpec((2, rows, 128), lambda i: (0, 0, 0)),
        ],
        out_specs=pl.BlockSpec((rows, 128), lambda i: (0, 0)),
        out_shape=jax.ShapeDtypeStruct((rows, 128), jnp.float32),
    )


def _norm_body(a0_ref, a1_ref, d0_ref, d1_ref, ent_ref, res_ref,
               oent_ref, ores_ref):
    dt = d0_ref[...] + d1_ref[...]
    dts = jnp.where(dt > 0.0, dt, 1.0)
    a = (a0_ref[...] + a1_ref[...]) / dts + ent_ref[...]
    nrm = jnp.sqrt(jnp.sum(a * a, axis=1, keepdims=True))
    ent_new = a / jnp.maximum(nrm, 1e-12)
    oent_ref[...] = ent_new
    ores_ref[...] = LAM * res_ref[...] + ent_new


@functools.lru_cache(maxsize=None)
def _make_norm(n, d, blk):
    grid = n // blk
    return pl.pallas_call(
        _norm_body,
        grid=(grid,),
        in_specs=[
            pl.BlockSpec((blk, d), lambda i: (i, 0)),
            pl.BlockSpec((blk, d), lambda i: (i, 0)),
            pl.BlockSpec((blk, 1), lambda i: (i, 0)),
            pl.BlockSpec((blk, 1), lambda i: (i, 0)),
            pl.BlockSpec((blk, d), lambda i: (i, 0)),
            pl.BlockSpec((blk, d), lambda i: (i, 0)),
        ],
        out_specs=[
            pl.BlockSpec((blk, d), lambda i: (i, 0)),
            pl.BlockSpec((blk, d), lambda i: (i, 0)),
        ],
        out_shape=[
            jax.ShapeDtypeStruct((n, d), jnp.float32),
            jax.ShapeDtypeStruct((n, d), jnp.float32),
        ],
    )


def _mesh():
    return plsc.VectorSubcoreMesh(core_axis_name="c", subcore_axis_name="s")


def _wid():
    return lax.axis_index("s") * 2 + lax.axis_index("c")


@functools.lru_cache(maxsize=None)
def _make_sc_logits(rows_w, r):
    nrows = NW * rows_w

    @functools.partial(
        pl.kernel, mesh=_mesh(),
        out_type=(jax.ShapeDtypeStruct((nrows, 128), jnp.float32),
                  jax.ShapeDtypeStruct((NW * LANES,), jnp.float32)),
        scratch_types=[
            pltpu.VMEM((rows_w, 128), jnp.int32),
            pltpu.VMEM((rows_w, 128), jnp.int32),
            pltpu.VMEM((rows_w, 128), jnp.int32),
            pltpu.VMEM((rows_w, 128), jnp.int32),
            pltpu.VMEM((rows_w, 128), jnp.int32),
            pltpu.VMEM((rows_w * 128,), jnp.float32),
            pltpu.VMEM((rows_w * 128,), jnp.float32),
            pltpu.VMEM((rows_w, 128), jnp.float32),
            pltpu.VMEM((LANES,), jnp.float32),
            pltpu.SemaphoreType.DMA,
            pltpu.SemaphoreType.DMA,
        ])
    def k(p1f, p2f, head2, tail2, type2, e_out, pmax_out,
          hbuf, tbuf, ybuf, i1, i2, v1, v2, ebuf, mbuf, sem1, sem2):
        w = _wid()
        base = w * rows_w
        pltpu.sync_copy(head2.at[pl.ds(base, rows_w)], hbuf)
        pltpu.sync_copy(tail2.at[pl.ds(base, rows_w)], tbuf)
        pltpu.sync_copy(type2.at[pl.ds(base, rows_w)], ybuf)

        def idx_row(j, c):
            for kk in range(8):
                sl = pl.ds(kk * LANES, LANES)
                y = ybuf[j, sl]
                i1[j, sl] = hbuf[j, sl] * r + y
                i2[j, sl] = tbuf[j, sl] * r + y
            return c
        lax.fori_loop(0, rows_w, idx_row, 0)

        def fire(j, c):
            pltpu.async_copy(p1f.at[i1.at[j]],
                             v1.at[pl.ds(j * 128, 128)], sem1)
            pltpu.async_copy(p2f.at[i2.at[j]],
                             v2.at[pl.ds(j * 128, 128)], sem2)
            return c
        lax.fori_loop(0, rows_w, fire, 0)
        pltpu.make_async_copy(p1f.at[pl.ds(0, rows_w * 128)], v1, sem1).wait()
        pltpu.make_async_copy(p2f.at[pl.ds(0, rows_w * 128)], v2, sem2).wait()

        def e_row(j, m):
            for kk in range(8):
                sl = pl.ds(kk * LANES, LANES)
                fsl = pl.ds(j * 128 + kk * LANES, LANES)
                s = v1[fsl] + v2[fsl]
                ev = jnp.where(s >= 0.0, s, NEG_SLOPE * s)
                ebuf[j, sl] = ev
                m = jnp.maximum(m, ev)
            return m
        m = lax.fori_loop(0, rows_w, e_row,
                          jnp.full((LANES,), -3e38, jnp.float32))
        mbuf[...] = m
        pltpu.sync_copy(ebuf, e_out.at[pl.ds(base, rows_w)])
        pltpu.sync_copy(mbuf, pmax_out.at[pl.ds(w * LANES, LANES)])

    return k


def _gmax_from(pv):
    def mrow(i, m):
        return jnp.maximum(m, pv[pl.ds(i * LANES, LANES)])
    m = lax.fori_loop(0, NW, mrow, jnp.full((LANES,), -3e38, jnp.float32))
    idx = lax.iota(jnp.int32, LANES)
    for s in (8, 4, 2, 1):
        m = jnp.maximum(m, m.at[idx ^ s].get(mode="promise_in_bounds"))
    return m[0]


@functools.lru_cache(maxsize=None)
def _make_sc_den1(rows_w, e_real, npad):
    stripe = npad // LANES

    @functools.partial(
        pl.kernel, mesh=_mesh(),
        out_type=jax.ShapeDtypeStruct((2 * npad,), jnp.float32),
        scratch_types=[
            pltpu.VMEM((rows_w, 128), jnp.float32),
            pltpu.VMEM((rows_w, 128), jnp.float32),
            pltpu.VMEM((rows_w, 128), jnp.int32),
            pltpu.VMEM((NW * LANES,), jnp.float32),
            pltpu.VMEM((npad // LANES,), jnp.float32),
            pltpu.VMEM_SHARED((npad,), jnp.float32),
        ])
    def k(e2, pmax, head2, den_out, ebuf, xbuf, hbuf, pv, zbuf, dsh):
        w = _wid()
        cid = lax.axis_index("c")
        sid = lax.axis_index("s")
        base = w * rows_w
        pltpu.sync_copy(e2.at[pl.ds(base, rows_w)], ebuf)
        pltpu.sync_copy(head2.at[pl.ds(base, rows_w)], hbuf)
        pltpu.sync_copy(pmax, pv)
        gmax = _gmax_from(pv)

        def zrow(q, c):
            zbuf[pl.ds(q * LANES, LANES)] = jnp.zeros((LANES,), jnp.float32)
            return c
        lax.fori_loop(0, stripe // LANES, zrow, 0)
        pltpu.sync_copy(zbuf, dsh.at[pl.ds(sid * stripe, stripe)])
        plsc.subcore_barrier()

        iota = lax.iota(jnp.int32, LANES)

        def x_row(j, c):
            gid0 = (base + j) * 128
            for kk in range(8):
                sl = pl.ds(kk * LANES, LANES)
                x = jnp.exp(ebuf[j, sl] - gmax)
                gid = gid0 + kk * LANES + iota
                xbuf[j, sl] = jnp.where(gid < e_real, x, 0.0)
            return c
        lax.fori_loop(0, rows_w, x_row, 0)

        def scat(j, c):
            pltpu.sync_copy(xbuf.at[j], dsh.at[hbuf.at[j]], add=True)
            return c
        lax.fori_loop(0, rows_w, scat, 0)
        plsc.subcore_barrier()
        pltpu.sync_copy(dsh.at[pl.ds(sid * stripe, stripe)],
                        den_out.at[pl.ds(cid * npad + sid * stripe, stripe)])

    return k


@functools.lru_cache(maxsize=None)
def _make_sc_den2(rows_w, e_real, npad):
    nrows = NW * rows_w
    stripe = npad // LANES

    @functools.partial(
        pl.kernel, mesh=_mesh(),
        out_type=(jax.ShapeDtypeStruct((nrows, 128), jnp.float32),
                  jax.ShapeDtypeStruct((2 * npad,), jnp.float32)),
        scratch_types=[
            pltpu.VMEM((rows_w, 128), jnp.float32),
            pltpu.VMEM((rows_w, 128), jnp.float32),
            pltpu.VMEM((rows_w, 128), jnp.int32),
            pltpu.VMEM((rows_w * 128,), jnp.float32),
            pltpu.VMEM((npad // LANES,), jnp.float32),
            pltpu.VMEM_SHARED((npad,), jnp.float32),
            pltpu.SemaphoreType.DMA,
        ])
    def k(e2, s1d, head2, ex_out, den_out,
          ebuf, xbuf, hbuf, sv, zbuf, dsh, sem):
        w = _wid()
        cid = lax.axis_index("c")
        sid = lax.axis_index("s")
        base = w * rows_w
        pltpu.sync_copy(e2.at[pl.ds(base, rows_w)], ebuf)
        pltpu.sync_copy(head2.at[pl.ds(base, rows_w)], hbuf)

        def fire(j, c):
            pltpu.async_copy(s1d.at[hbuf.at[j]],
                             sv.at[pl.ds(j * 128, 128)], sem)
            return c
        lax.fori_loop(0, rows_w, fire, 0)

        def zrow(q, c):
            zbuf[pl.ds(q * LANES, LANES)] = jnp.zeros((LANES,), jnp.float32)
            return c
        lax.fori_loop(0, stripe // LANES, zrow, 0)
        pltpu.sync_copy(zbuf, dsh.at[pl.ds(sid * stripe, stripe)])
        pltpu.make_async_copy(s1d.at[pl.ds(0, rows_w * 128)], sv, sem).wait()
        plsc.subcore_barrier()

        iota = lax.iota(jnp.int32, LANES)

        def x_row(j, c):
            gid0 = (base + j) * 128
            for kk in range(8):
                sl = pl.ds(kk * LANES, LANES)
                fsl = pl.ds(j * 128 + kk * LANES, LANES)
                x = jnp.exp(jnp.minimum(ebuf[j, sl] - sv[fsl], 1.0))
                gid = gid0 + kk * LANES + iota
                xbuf[j, sl] = jnp.where(gid < e_real, x, 0.0)
            return c
        lax.fori_loop(0, rows_w, x_row, 0)

        pltpu.sync_copy(xbuf, ex_out.at[pl.ds(base, rows_w)])

        def scat(j, c):
            pltpu.sync_copy(xbuf.at[j], dsh.at[hbuf.at[j]], add=True)
            return c
        lax.fori_loop(0, rows_w, scat, 0)
        plsc.subcore_barrier()
        pltpu.sync_copy(dsh.at[pl.ds(sid * stripe, stripe)],
                        den_out.at[pl.ds(cid * npad + sid * stripe, stripe)])

    return k


@functools.lru_cache(maxsize=None)
def _make_sc_agg(rows_w, npad, d):
    stripe = npad // LANES

    @functools.partial(
        pl.kernel, mesh=_mesh(),
        out_type=jax.ShapeDtypeStruct((2, npad, d), jnp.float32),
        scratch_types=[
            pltpu.VMEM((rows_w, 128), jnp.int32),
            pltpu.VMEM((rows_w, 128), jnp.int32),
            pltpu.VMEM((rows_w, 128), jnp.float32),
            pltpu.VMEM((128, d), jnp.float32),
            pltpu.VMEM_SHARED((npad, d), jnp.float32),
            pltpu.SemaphoreType.DMA,
        ])
    def k(ex2, head2, tail2, ent, agg_out,
          hbuf, tbuf, xbuf, rows, ash, sem):
        w = _wid()
        cid = lax.axis_index("c")
        sid = lax.axis_index("s")
        base = w * rows_w
        pltpu.sync_copy(head2.at[pl.ds(base, rows_w)], hbuf)
        pltpu.sync_copy(tail2.at[pl.ds(base, rows_w)], tbuf)
        pltpu.sync_copy(ex2.at[pl.ds(base, rows_w)], xbuf)

        def zr(i, c):
            for kk in range(d // LANES):
                rows[i, pl.ds(kk * LANES, LANES)] = jnp.zeros((LANES,),
                                                              jnp.float32)
            return c
        lax.fori_loop(0, 128, zr, 0)

        def zcopy(q, c):
            pltpu.sync_copy(rows, ash.at[pl.ds(sid * stripe + q * 128, 128)])
            return c
        lax.fori_loop(0, stripe // 128, zcopy, 0)
        plsc.subcore_barrier()

        def chunk(j, c):
            pltpu.async_copy(ent.at[tbuf.at[j]], rows, sem).wait()

            def srow_g(g, cc):
                a16 = xbuf[j, pl.ds(g * LANES, LANES)]
                for ll in range(LANES):
                    i = g * LANES + ll
                    a = a16[ll]
                    for kk in range(d // LANES):
                        sl = pl.ds(kk * LANES, LANES)
                        rows[i, sl] = rows[i, sl] * a
                return cc
            lax.fori_loop(0, 8, srow_g, 0)
            pltpu.sync_copy(rows, ash.at[hbuf.at[j]], add=True)
            return c
        lax.fori_loop(0, rows_w, chunk, 0)
        plsc.subcore_barrier()

        pltpu.sync_copy(ash.at[pl.ds(sid * stripe, stripe)],
                        agg_out.at[cid, pl.ds(sid * stripe, stripe)])

    return k


def kernel(entity_emb, relation_emb, edge_index, edge_type, W):
    n, d = entity_emb.shape
    r = relation_emb.shape[0]
    e_real = edge_type.shape[0]
    rows_w = _rup(-(-e_real // (NW * 128)), 8)
    epad = NW * rows_w * 128
    npad = _rup(n, LANES * 128)
    blk = 1000 if n % 1000 == 0 else 8

    head = edge_index[0].astype(jnp.int32)
    tail = edge_index[1].astype(jnp.int32)
    etype = edge_type.astype(jnp.int32)
    pad = epad - e_real
    zpad = jnp.zeros((pad,), jnp.int32)
    head2 = jnp.concatenate([head, zpad]).reshape(NW * rows_w, 128)
    tail2 = jnp.concatenate([tail, zpad]).reshape(NW * rows_w, 128)
    type2 = jnp.concatenate([etype, zpad]).reshape(NW * rows_w, 128)

    proj = _make_proj(n, d, r, blk)
    shift = _make_shift(npad)
    norm = _make_norm(n, d, blk)
    sc_a = _make_sc_logits(rows_w, r)
    sc_b1 = _make_sc_den1(rows_w, e_real, npad)
    sc_b2 = _make_sc_den2(rows_w, e_real, npad)
    sc_c = _make_sc_agg(rows_w, npad, d)

    ent = entity_emb
    res = entity_emb
    for _ in range(N_HOPS):
        p1, p2 = proj(ent, W, relation_emb)
        p1f = p1.reshape(n * r)
        p2f = p2.reshape(n * r)
        e2, pmax = sc_a(p1f, p2f, head2, tail2, type2)
        den1 = sc_b1(e2, pmax, head2)
        s2 = shift(pmax.reshape(NW, LANES),
                   den1.reshape(2, npad // 128, 128)).reshape(npad)
        ex2, den2 = sc_b2(e2, s2, head2)
        agg = sc_c(ex2, head2, tail2, ent)
        d0c = den2[:npad].reshape(npad, 1)
        d1c = den2[npad:].reshape(npad, 1)
        ent, res = norm(agg[0, :n], agg[1, :n], d0c[:n], d1c[:n], ent, res)
    return res


# speedup vs baseline: 8.5461x; 1.0264x over previous
import functools

import jax
import jax.numpy as jnp
from jax import lax
from jax.experimental import pallas as pl
from jax.experimental.pallas import tpu as pltpu
from jax.experimental.pallas import tpu_sc as plsc

NEG_SLOPE = 0.2
LAM = 0.5
N_HOPS = 2
NW = 32
LANES = 16


def _rup(x, m):
    return (x + m - 1) // m * m


def _proj_body(ent_ref, w_ref, rel_ref, p1_ref, p2_ref):
    d = ent_ref.shape[1]
    cdims = (((1,), (1,)), ((), ()))
    m1 = lax.dot_general(w_ref[0:d, :], rel_ref[...], cdims,
                         preferred_element_type=jnp.float32)
    m2 = lax.dot_general(w_ref[d:2 * d, :], rel_ref[...], cdims,
                         preferred_element_type=jnp.float32)
    e = ent_ref[...]
    p1_ref[...] = jnp.dot(e, m1, preferred_element_type=jnp.float32)
    p2_ref[...] = jnp.dot(e, m2, preferred_element_type=jnp.float32)


@functools.lru_cache(maxsize=None)
def _make_proj(n, d, r, blk):
    grid = n // blk
    return pl.pallas_call(
        _proj_body,
        grid=(grid,),
        in_specs=[
            pl.BlockSpec((blk, d), lambda i: (i, 0)),
            pl.BlockSpec((2 * d, d), lambda i: (0, 0)),
            pl.BlockSpec((r, d), lambda i: (0, 0)),
        ],
        out_specs=[
            pl.BlockSpec((blk, r), lambda i: (i, 0)),
            pl.BlockSpec((blk, r), lambda i: (i, 0)),
        ],
        out_shape=[
            jax.ShapeDtypeStruct((n, r), jnp.float32),
            jax.ShapeDtypeStruct((n, r), jnp.float32),
        ],
    )


def _shift_body(msc_ref, den_ref, s_ref):
    m0 = msc_ref[0, 0]
    m1 = msc_ref[1, 0]
    g = jnp.maximum(m0, m1)
    dt = jnp.exp(m0 - g) * den_ref[0] + jnp.exp(m1 - g) * den_ref[1]
    dts = jnp.where(dt > 0.0, dt, 1.0)
    s_ref[...] = jnp.where(dt > 0.0, g + jnp.log(dts), g - 88.0)


@functools.lru_cache(maxsize=None)
def _make_shift(npad):
    rows = npad // 128
    return pl.pallas_call(
        _shift_body,
        grid=(1,),
        in_specs=[
            pl.BlockSpec((2, LANES), lambda i: (0, 0)),
            pl.BlockSpec((2, rows, 128), lambda i: (0, 0, 0)),
        ],
        out_specs=pl.BlockSpec((rows, 128), lambda i: (0, 0)),
        out_shape=jax.ShapeDtypeStruct((rows, 128), jnp.float32),
    )


def _norm_body(a0_ref, a1_ref, d0_ref, d1_ref, ent_ref, res_ref,
               oent_ref, ores_ref):
    dt = d0_ref[...] + d1_ref[...]
    dts = jnp.where(dt > 0.0, dt, 1.0)
    a = (a0_ref[...] + a1_ref[...]) / dts + ent_ref[...]
    nrm = jnp.sqrt(jnp.sum(a * a, axis=1, keepdims=True))
    ent_new = a / jnp.maximum(nrm, 1e-12)
    oent_ref[...] = ent_new
    ores_ref[...] = LAM * res_ref[...] + ent_new


@functools.lru_cache(maxsize=None)
def _make_norm(n, d, blk):
    grid = n // blk
    return pl.pallas_call(
        _norm_body,
        grid=(grid,),
        in_specs=[
            pl.BlockSpec((blk, d), lambda i: (i, 0)),
            pl.BlockSpec((blk, d), lambda i: (i, 0)),
            pl.BlockSpec((blk, 1), lambda i: (i, 0)),
            pl.BlockSpec((blk, 1), lambda i: (i, 0)),
            pl.BlockSpec((blk, d), lambda i: (i, 0)),
            pl.BlockSpec((blk, d), lambda i: (i, 0)),
        ],
        out_specs=[
            pl.BlockSpec((blk, d), lambda i: (i, 0)),
            pl.BlockSpec((blk, d), lambda i: (i, 0)),
        ],
        out_shape=[
            jax.ShapeDtypeStruct((n, d), jnp.float32),
            jax.ShapeDtypeStruct((n, d), jnp.float32),
        ],
    )


def _mesh():
    return plsc.VectorSubcoreMesh(core_axis_name="c", subcore_axis_name="s")


def _wid():
    return lax.axis_index("s") * 2 + lax.axis_index("c")


def _butterfly_max(m):
    idx = lax.iota(jnp.int32, LANES)
    for s in (8, 4, 2, 1):
        m = jnp.maximum(m, m.at[idx ^ s].get(mode="promise_in_bounds"))
    return m[0]


@functools.lru_cache(maxsize=None)
def _make_sc_logits_den(rows_w, r, e_real, npad):
    nrows = NW * rows_w
    stripe = npad // LANES

    @functools.partial(
        pl.kernel, mesh=_mesh(),
        out_type=(jax.ShapeDtypeStruct((nrows, 128), jnp.float32),
                  jax.ShapeDtypeStruct((2 * npad,), jnp.float32),
                  jax.ShapeDtypeStruct((2 * LANES,), jnp.float32)),
        scratch_types=[
            pltpu.VMEM((rows_w, 128), jnp.int32),
            pltpu.VMEM((rows_w, 128), jnp.int32),
            pltpu.VMEM((rows_w, 128), jnp.int32),
            pltpu.VMEM((rows_w, 128), jnp.int32),
            pltpu.VMEM((rows_w, 128), jnp.int32),
            pltpu.VMEM((rows_w * 128,), jnp.float32),
            pltpu.VMEM((rows_w * 128,), jnp.float32),
            pltpu.VMEM((rows_w, 128), jnp.float32),
            pltpu.VMEM((rows_w, 128), jnp.float32),
            pltpu.VMEM((LANES,), jnp.float32),
            pltpu.VMEM((LANES, LANES), jnp.float32),
            pltpu.VMEM((npad // LANES,), jnp.float32),
            pltpu.VMEM_SHARED((LANES, LANES), jnp.float32),
            pltpu.VMEM_SHARED((npad,), jnp.float32),
            pltpu.SemaphoreType.DMA,
            pltpu.SemaphoreType.DMA,
        ])
    def k(p1f, p2f, head2, tail2, type2, e_out, den_out, msc_out,
          hbuf, tbuf, ybuf, i1, i2, v1, v2, ebuf, xbuf, mbuf, pvb,
          zbuf, dshm, dsh, sem1, sem2):
        w = _wid()
        cid = lax.axis_index("c")
        sid = lax.axis_index("s")
        base = w * rows_w
        pltpu.sync_copy(head2.at[pl.ds(base, rows_w)], hbuf)
        pltpu.sync_copy(tail2.at[pl.ds(base, rows_w)], tbuf)
        pltpu.sync_copy(type2.at[pl.ds(base, rows_w)], ybuf)

        def idx_row(j, c):
            for kk in range(8):
                sl = pl.ds(kk * LANES, LANES)
                y = ybuf[j, sl]
                i1[j, sl] = hbuf[j, sl] * r + y
                i2[j, sl] = tbuf[j, sl] * r + y
            return c
        lax.fori_loop(0, rows_w, idx_row, 0)

        def fire(j, c):
            pltpu.async_copy(p1f.at[i1.at[j]],
                             v1.at[pl.ds(j * 128, 128)], sem1)
            pltpu.async_copy(p2f.at[i2.at[j]],
                             v2.at[pl.ds(j * 128, 128)], sem2)
            return c
        lax.fori_loop(0, rows_w, fire, 0)

        def zrow(q, c):
            zbuf[pl.ds(q * LANES, LANES)] = jnp.zeros((LANES,), jnp.float32)
            return c
        lax.fori_loop(0, stripe // LANES, zrow, 0)
        pltpu.sync_copy(zbuf, dsh.at[pl.ds(sid * stripe, stripe)])

        pltpu.make_async_copy(p1f.at[pl.ds(0, rows_w * 128)], v1, sem1).wait()
        pltpu.make_async_copy(p2f.at[pl.ds(0, rows_w * 128)], v2, sem2).wait()

        def e_row(j, m):
            for kk in range(8):
                sl = pl.ds(kk * LANES, LANES)
                fsl = pl.ds(j * 128 + kk * LANES, LANES)
                s = v1[fsl] + v2[fsl]
                ev = jnp.where(s >= 0.0, s, NEG_SLOPE * s)
                ebuf[j, sl] = ev
                m = jnp.maximum(m, ev)
            return m
        m = lax.fori_loop(0, rows_w, e_row,
                          jnp.full((LANES,), -3e38, jnp.float32))
        mbuf[...] = m
        pltpu.sync_copy(ebuf, e_out.at[pl.ds(base, rows_w)])

        pltpu.sync_copy(mbuf, dshm.at[sid])
        plsc.subcore_barrier()
        pltpu.sync_copy(dshm, pvb)

        def mrow(i, mm):
            return jnp.maximum(mm, pvb[i])
        mc = _butterfly_max(lax.fori_loop(
            0, LANES, mrow, jnp.full((LANES,), -3e38, jnp.float32)))

        iota = lax.iota(jnp.int32, LANES)

        def x_row(j, c):
            gid0 = (base + j) * 128
            for kk in range(8):
                sl = pl.ds(kk * LANES, LANES)
                x = jnp.exp(ebuf[j, sl] - mc)
                gid = gid0 + kk * LANES + iota
                xbuf[j, sl] = jnp.where(gid < e_real, x, 0.0)
            return c
        lax.fori_loop(0, rows_w, x_row, 0)

        def scat(j, c):
            pltpu.sync_copy(xbuf.at[j], dsh.at[hbuf.at[j]], add=True)
            return c
        lax.fori_loop(0, rows_w, scat, 0)
        plsc.subcore_barrier()
        pltpu.sync_copy(dsh.at[pl.ds(sid * stripe, stripe)],
                        den_out.at[pl.ds(cid * npad + sid * stripe, stripe)])

        @pl.when(sid == 0)
        def _():
            mbuf[...] = jnp.full((LANES,), 0.0, jnp.float32) + mc
            pltpu.sync_copy(mbuf, msc_out.at[pl.ds(cid * LANES, LANES)])

    return k


@functools.lru_cache(maxsize=None)
def _make_sc_aggden(rows_w, e_real, npad, d):
    stripe = npad // LANES
    ngrp = rows_w // 8

    @functools.partial(
        pl.kernel, mesh=_mesh(),
        out_type=(jax.ShapeDtypeStruct((2, npad, d), jnp.float32),
                  jax.ShapeDtypeStruct((2 * npad,), jnp.float32)),
        scratch_types=[
            pltpu.VMEM((rows_w, 128), jnp.int32),
            pltpu.VMEM((rows_w, 128), jnp.int32),
            pltpu.VMEM((rows_w, 128), jnp.float32),
            pltpu.VMEM((8 * 128,), jnp.float32),
            pltpu.VMEM((128, d), jnp.float32),
            pltpu.VMEM_SHARED((npad, d), jnp.float32),
            pltpu.VMEM_SHARED((npad,), jnp.float32),
            pltpu.SemaphoreType.DMA,
            pltpu.SemaphoreType.DMA,
        ])
    def k(e2, s1d, head2, tail2, ent, agg_out, den_out,
          hbuf, tbuf, xbuf, sv, rows, ash, dsh, semx, sema):
        w = _wid()
        cid = lax.axis_index("c")
        sid = lax.axis_index("s")
        base = w * rows_w
        pltpu.sync_copy(head2.at[pl.ds(base, rows_w)], hbuf)
        pltpu.sync_copy(tail2.at[pl.ds(base, rows_w)], tbuf)
        pltpu.sync_copy(e2.at[pl.ds(base, rows_w)], xbuf)

        def zr(i, c):
            for kk in range(d // LANES):
                rows[i, pl.ds(kk * LANES, LANES)] = jnp.zeros(
                    (LANES,), jnp.float32)
            return c
        lax.fori_loop(0, 128, zr, 0)

        def zcopy(q, c):
            pltpu.sync_copy(rows, ash.at[pl.ds(sid * stripe + q * 128, 128)])
            return c
        lax.fori_loop(0, stripe // 128, zcopy, 0)

        def zs(q, c):
            sv[pl.ds(q * LANES, LANES)] = jnp.zeros((LANES,), jnp.float32)
            return c
        lax.fori_loop(0, (8 * 128) // LANES, zs, 0)
        pltpu.sync_copy(sv.at[pl.ds(0, stripe)],
                        dsh.at[pl.ds(sid * stripe, stripe)])
        plsc.subcore_barrier()

        iota = lax.iota(jnp.int32, LANES)

        def grp(g, c):
            def fire_s(q, cc):
                pltpu.async_copy(s1d.at[hbuf.at[g * 8 + q]],
                                 sv.at[pl.ds(q * 128, 128)], semx)
                return cc
            lax.fori_loop(0, 8, fire_s, 0)
            pltpu.make_async_copy(s1d.at[pl.ds(0, 8 * 128)], sv,
                                  semx).wait()

            def x_row(q, cc):
                j = g * 8 + q
                gid0 = (base + j) * 128
                for kk in range(8):
                    sl = pl.ds(kk * LANES, LANES)
                    fsl = pl.ds(q * 128 + kk * LANES, LANES)
                    x = jnp.exp(jnp.minimum(xbuf[j, sl] - sv[fsl], 1.0))
                    gid = gid0 + kk * LANES + iota
                    xbuf[j, sl] = jnp.where(gid < e_real, x, 0.0)
                return cc
            lax.fori_loop(0, 8, x_row, 0)

            def dscat(q, cc):
                j = g * 8 + q
                pltpu.sync_copy(xbuf.at[j], dsh.at[hbuf.at[j]], add=True)
                return cc
            lax.fori_loop(0, 8, dscat, 0)
            return c
        lax.fori_loop(0, ngrp, grp, 0)

        def scale(j):
            def srow_g(g, cc):
                a16 = xbuf[j, pl.ds(g * LANES, LANES)]
                for ll in range(LANES):
                    i = g * LANES + ll
                    a = a16[ll]
                    for kk in range(d // LANES):
                        sl = pl.ds(kk * LANES, LANES)
                        rows[i, sl] = rows[i, sl] * a
                return cc
            lax.fori_loop(0, 8, srow_g, 0)

        def pipe(j, c):
            pltpu.async_copy(ent.at[tbuf.at[j]], rows, sema).wait()
            scale(j)
            pltpu.sync_copy(rows, ash.at[hbuf.at[j]], add=True)
            return c
        lax.fori_loop(0, rows_w, pipe, 0)
        plsc.subcore_barrier()

        pltpu.sync_copy(ash.at[pl.ds(sid * stripe, stripe)],
                        agg_out.at[cid, pl.ds(sid * stripe, stripe)])
        pltpu.sync_copy(dsh.at[pl.ds(sid * stripe, stripe)],
                        den_out.at[pl.ds(cid * npad + sid * stripe, stripe)])

    return k


def kernel(entity_emb, relation_emb, edge_index, edge_type, W):
    n, d = entity_emb.shape
    r = relation_emb.shape[0]
    e_real = edge_type.shape[0]
    rows_w = _rup(-(-e_real // (NW * 128)), 8)
    epad = NW * rows_w * 128
    npad = _rup(n, LANES * 128)
    blk = 1000 if n % 1000 == 0 else 8

    head = edge_index[0].astype(jnp.int32)
    tail = edge_index[1].astype(jnp.int32)
    etype = edge_type.astype(jnp.int32)
    pad = epad - e_real
    zpad = jnp.zeros((pad,), jnp.int32)
    head2 = jnp.concatenate([head, zpad]).reshape(NW * rows_w, 128)
    tail2 = jnp.concatenate([tail, zpad]).reshape(NW * rows_w, 128)
    type2 = jnp.concatenate([etype, zpad]).reshape(NW * rows_w, 128)

    proj = _make_proj(n, d, r, blk)
    shift = _make_shift(npad)
    norm = _make_norm(n, d, blk)
    sc_a = _make_sc_logits_den(rows_w, r, e_real, npad)
    sc_c = _make_sc_aggden(rows_w, e_real, npad, d)

    ent = entity_emb
    res = entity_emb
    for _ in range(N_HOPS):
        p1, p2 = proj(ent, W, relation_emb)
        p1f = p1.reshape(n * r)
        p2f = p2.reshape(n * r)
        e2, den1, msc = sc_a(p1f, p2f, head2, tail2, type2)
        s2 = shift(msc.reshape(2, LANES),
                   den1.reshape(2, npad // 128, 128)).reshape(npad)
        agg, den2 = sc_c(e2, s2, head2, tail2, ent)
        d0c = den2[:npad].reshape(npad, 1)
        d1c = den2[npad:].reshape(npad, 1)
        ent, res = norm(agg[0, :n], agg[1, :n], d0c[:n], d1c[:n], ent, res)
    return res


# speedup vs baseline: 8.5512x; 1.0006x over previous
import functools

import jax
import jax.numpy as jnp
from jax import lax
from jax.experimental import pallas as pl
from jax.experimental.pallas import tpu as pltpu
from jax.experimental.pallas import tpu_sc as plsc

NEG_SLOPE = 0.2
LAM = 0.5
N_HOPS = 2
NW = 32
LANES = 16


def _rup(x, m):
    return (x + m - 1) // m * m


def _proj_body(ent_ref, w_ref, rel_ref, p1_ref, p2_ref):
    d = ent_ref.shape[1]
    cdims = (((1,), (1,)), ((), ()))
    m1 = lax.dot_general(w_ref[0:d, :], rel_ref[...], cdims,
                         preferred_element_type=jnp.float32)
    m2 = lax.dot_general(w_ref[d:2 * d, :], rel_ref[...], cdims,
                         preferred_element_type=jnp.float32)
    e = ent_ref[...]
    p1_ref[...] = jnp.dot(e, m1, preferred_element_type=jnp.float32)
    p2_ref[...] = jnp.dot(e, m2, preferred_element_type=jnp.float32)


@functools.lru_cache(maxsize=None)
def _make_proj(n, d, r, blk):
    grid = n // blk
    return pl.pallas_call(
        _proj_body,
        grid=(grid,),
        in_specs=[
            pl.BlockSpec((blk, d), lambda i: (i, 0)),
            pl.BlockSpec((2 * d, d), lambda i: (0, 0)),
            pl.BlockSpec((r, d), lambda i: (0, 0)),
        ],
        out_specs=[
            pl.BlockSpec((blk, r), lambda i: (i, 0)),
            pl.BlockSpec((blk, r), lambda i: (i, 0)),
        ],
        out_shape=[
            jax.ShapeDtypeStruct((n, r), jnp.float32),
            jax.ShapeDtypeStruct((n, r), jnp.float32),
        ],
    )


def _shift_body(pm0_ref, pm1_ref, den_ref, s_ref):
    m0 = jnp.max(pm0_ref[...])
    m1 = jnp.max(pm1_ref[...])
    g = jnp.maximum(m0, m1)
    dt = jnp.exp(m0 - g) * den_ref[0] + jnp.exp(m1 - g) * den_ref[1]
    dts = jnp.where(dt > 0.0, dt, 1.0)
    s_ref[...] = jnp.where(dt > 0.0, g + jnp.log(dts), g - 88.0)


@functools.lru_cache(maxsize=None)
def _make_shift(npad):
    rows = npad // 128
    return pl.pallas_call(
        _shift_body,
        grid=(1,),
        in_specs=[
            pl.BlockSpec((LANES, LANES), lambda i: (0, 0)),
            pl.BlockSpec((LANES, LANES), lambda i: (0, 0)),
            pl.BlockSpec((2, rows, 128), lambda i: (0, 0, 0)),
        ],
        out_specs=pl.BlockSpec((rows, 128), lambda i: (0, 0)),
        out_shape=jax.ShapeDtypeStruct((rows, 128), jnp.float32),
    )


def _norm_body(a0_ref, a1_ref, d0_ref, d1_ref, ent_ref, res_ref,
               oent_ref, ores_ref):
    dt = d0_ref[...] + d1_ref[...]
    dts = jnp.where(dt > 0.0, dt, 1.0)
    a = (a0_ref[...] + a1_ref[...]) / dts + ent_ref[...]
    nrm = jnp.sqrt(jnp.sum(a * a, axis=1, keepdims=True))
    ent_new = a / jnp.maximum(nrm, 1e-12)
    oent_ref[...] = ent_new
    ores_ref[...] = LAM * res_ref[...] + ent_new


@functools.lru_cache(maxsize=None)
def _make_norm(n, d, blk):
    grid = n // blk
    return pl.pallas_call(
        _norm_body,
        grid=(grid,),
        in_specs=[
            pl.BlockSpec((blk, d), lambda i: (i, 0)),
            pl.BlockSpec((blk, d), lambda i: (i, 0)),
            pl.BlockSpec((blk, 1), lambda i: (i, 0)),
            pl.BlockSpec((blk, 1), lambda i: (i, 0)),
            pl.BlockSpec((blk, d), lambda i: (i, 0)),
            pl.BlockSpec((blk, d), lambda i: (i, 0)),
        ],
        out_specs=[
            pl.BlockSpec((blk, d), lambda i: (i, 0)),
            pl.BlockSpec((blk, d), lambda i: (i, 0)),
        ],
        out_shape=[
            jax.ShapeDtypeStruct((n, d), jnp.float32),
            jax.ShapeDtypeStruct((n, d), jnp.float32),
        ],
    )


def _mesh():
    return plsc.VectorSubcoreMesh(core_axis_name="c", subcore_axis_name="s")


def _wid():
    return lax.axis_index("s") * 2 + lax.axis_index("c")


def _butterfly_max(m):
    idx = lax.iota(jnp.int32, LANES)
    for s in (8, 4, 2, 1):
        m = jnp.maximum(m, m.at[idx ^ s].get(mode="promise_in_bounds"))
    return m[0]


@functools.lru_cache(maxsize=None)
def _make_sc_logits_den(rows_w, r, e_real, npad):
    nrows = NW * rows_w
    stripe = npad // LANES

    @functools.partial(
        pl.kernel, mesh=_mesh(),
        out_type=(jax.ShapeDtypeStruct((nrows, 128), jnp.float32),
                  jax.ShapeDtypeStruct((2 * npad,), jnp.float32),
                  jax.ShapeDtypeStruct((NW * LANES,), jnp.float32)),
        scratch_types=[
            pltpu.VMEM((rows_w, 128), jnp.int32),
            pltpu.VMEM((rows_w, 128), jnp.int32),
            pltpu.VMEM((rows_w, 128), jnp.int32),
            pltpu.VMEM((rows_w, 128), jnp.int32),
            pltpu.VMEM((rows_w, 128), jnp.int32),
            pltpu.VMEM((rows_w * 128,), jnp.float32),
            pltpu.VMEM((rows_w * 128,), jnp.float32),
            pltpu.VMEM((rows_w, 128), jnp.float32),
            pltpu.VMEM((rows_w, 128), jnp.float32),
            pltpu.VMEM((LANES,), jnp.float32),
            pltpu.VMEM((LANES * LANES,), jnp.float32),
            pltpu.VMEM((npad // LANES,), jnp.float32),
            pltpu.VMEM_SHARED((LANES * LANES,), jnp.float32),
            pltpu.VMEM_SHARED((npad,), jnp.float32),
            pltpu.SemaphoreType.DMA,
            pltpu.SemaphoreType.DMA,
        ])
    def k(p1f, p2f, head2, tail2, type2, e_out, den_out, pmax_out,
          hbuf, tbuf, ybuf, i1, i2, v1, v2, ebuf, xbuf, mbuf, pvb,
          zbuf, dshm, dsh, sem1, sem2):
        w = _wid()
        cid = lax.axis_index("c")
        sid = lax.axis_index("s")
        base = w * rows_w
        pltpu.sync_copy(head2.at[pl.ds(base, rows_w)], hbuf)
        pltpu.sync_copy(tail2.at[pl.ds(base, rows_w)], tbuf)
        pltpu.sync_copy(type2.at[pl.ds(base, rows_w)], ybuf)

        def idx_row(j, c):
            for kk in range(8):
                sl = pl.ds(kk * LANES, LANES)
                y = ybuf[j, sl]
                i1[j, sl] = hbuf[j, sl] * r + y
                i2[j, sl] = tbuf[j, sl] * r + y
            return c
        lax.fori_loop(0, rows_w, idx_row, 0)

        def fire(j, c):
            pltpu.async_copy(p1f.at[i1.at[j]],
                             v1.at[pl.ds(j * 128, 128)], sem1)
            pltpu.async_copy(p2f.at[i2.at[j]],
                             v2.at[pl.ds(j * 128, 128)], sem2)
            return c
        lax.fori_loop(0, rows_w, fire, 0)

        def zrow(q, c):
            zbuf[pl.ds(q * LANES, LANES)] = jnp.zeros((LANES,), jnp.float32)
            return c
        lax.fori_loop(0, stripe // LANES, zrow, 0)
        pltpu.sync_copy(zbuf, dsh.at[pl.ds(sid * stripe, stripe)])

        pltpu.make_async_copy(p1f.at[pl.ds(0, rows_w * 128)], v1, sem1).wait()
        pltpu.make_async_copy(p2f.at[pl.ds(0, rows_w * 128)], v2, sem2).wait()

        def e_row(j, m):
            for kk in range(8):
                sl = pl.ds(kk * LANES, LANES)
                fsl = pl.ds(j * 128 + kk * LANES, LANES)
                s = v1[fsl] + v2[fsl]
                ev = jnp.where(s >= 0.0, s, NEG_SLOPE * s)
                ebuf[j, sl] = ev
                m = jnp.maximum(m, ev)
            return m
        m = lax.fori_loop(0, rows_w, e_row,
                          jnp.full((LANES,), -3e38, jnp.float32))
        mbuf[...] = m
        pltpu.sync_copy(ebuf, e_out.at[pl.ds(base, rows_w)])
        pltpu.sync_copy(mbuf, pmax_out.at[pl.ds(w * LANES, LANES)])

        pltpu.sync_copy(mbuf, dshm.at[pl.ds(sid * LANES, LANES)])
        plsc.subcore_barrier()
        pltpu.sync_copy(dshm, pvb)

        def mrow(i, mm):
            return jnp.maximum(mm, pvb[pl.ds(i * LANES, LANES)])
        mc = _butterfly_max(lax.fori_loop(
            0, LANES, mrow, jnp.full((LANES,), -3e38, jnp.float32)))

        iota = lax.iota(jnp.int32, LANES)

        def x_row(j, c):
            gid0 = (base + j) * 128
            for kk in range(8):
                sl = pl.ds(kk * LANES, LANES)
                x = jnp.exp(ebuf[j, sl] - mc)
                gid = gid0 + kk * LANES + iota
                xbuf[j, sl] = jnp.where(gid < e_real, x, 0.0)
            return c
        lax.fori_loop(0, rows_w, x_row, 0)

        def scat(j, c):
            pltpu.sync_copy(xbuf.at[j], dsh.at[hbuf.at[j]], add=True)
            return c
        lax.fori_loop(0, rows_w, scat, 0)
        plsc.subcore_barrier()
        pltpu.sync_copy(dsh.at[pl.ds(sid * stripe, stripe)],
                        den_out.at[pl.ds(cid * npad + sid * stripe, stripe)])

    return k


@functools.lru_cache(maxsize=None)
def _make_sc_aggden(rows_w, e_real, npad, d):
    stripe = npad // LANES
    ngrp = rows_w // 8

    @functools.partial(
        pl.kernel, mesh=_mesh(),
        out_type=(jax.ShapeDtypeStruct((2, npad, d), jnp.float32),
                  jax.ShapeDtypeStruct((2 * npad,), jnp.float32)),
        scratch_types=[
            pltpu.VMEM((rows_w, 128), jnp.int32),
            pltpu.VMEM((rows_w, 128), jnp.int32),
            pltpu.VMEM((rows_w, 128), jnp.float32),
            pltpu.VMEM((8 * 128,), jnp.float32),
            pltpu.VMEM((128, d), jnp.float32),
            pltpu.VMEM_SHARED((npad, d), jnp.float32),
            pltpu.VMEM_SHARED((npad,), jnp.float32),
            pltpu.SemaphoreType.DMA,
            pltpu.SemaphoreType.DMA,
        ])
    def k(e2, s1d, head2, tail2, ent, agg_out, den_out,
          hbuf, tbuf, xbuf, sv, rows, ash, dsh, semx, sema):
        w = _wid()
        cid = lax.axis_index("c")
        sid = lax.axis_index("s")
        base = w * rows_w
        pltpu.sync_copy(head2.at[pl.ds(base, rows_w)], hbuf)
        pltpu.sync_copy(tail2.at[pl.ds(base, rows_w)], tbuf)
        pltpu.sync_copy(e2.at[pl.ds(base, rows_w)], xbuf)

        def zr(i, c):
            for kk in range(d // LANES):
                rows[i, pl.ds(kk * LANES, LANES)] = jnp.zeros(
                    (LANES,), jnp.float32)
            return c
        lax.fori_loop(0, 128, zr, 0)

        def zcopy(q, c):
            pltpu.sync_copy(rows, ash.at[pl.ds(sid * stripe + q * 128, 128)])
            return c
        lax.fori_loop(0, stripe // 128, zcopy, 0)

        def zs(q, c):
            sv[pl.ds(q * LANES, LANES)] = jnp.zeros((LANES,), jnp.float32)
            return c
        lax.fori_loop(0, (8 * 128) // LANES, zs, 0)
        pltpu.sync_copy(sv.at[pl.ds(0, stripe)],
                        dsh.at[pl.ds(sid * stripe, stripe)])
        plsc.subcore_barrier()

        iota = lax.iota(jnp.int32, LANES)

        def grp(g, c):
            def fire_s(q, cc):
                pltpu.async_copy(s1d.at[hbuf.at[g * 8 + q]],
                                 sv.at[pl.ds(q * 128, 128)], semx)
                return cc
            lax.fori_loop(0, 8, fire_s, 0)
            pltpu.make_async_copy(s1d.at[pl.ds(0, 8 * 128)], sv,
                                  semx).wait()

            def x_row(q, cc):
                j = g * 8 + q
                gid0 = (base + j) * 128
                for kk in range(8):
                    sl = pl.ds(kk * LANES, LANES)
                    fsl = pl.ds(q * 128 + kk * LANES, LANES)
                    x = jnp.exp(jnp.minimum(xbuf[j, sl] - sv[fsl], 1.0))
                    gid = gid0 + kk * LANES + iota
                    xbuf[j, sl] = jnp.where(gid < e_real, x, 0.0)
                return cc
            lax.fori_loop(0, 8, x_row, 0)

            def dscat(q, cc):
                j = g * 8 + q
                pltpu.sync_copy(xbuf.at[j], dsh.at[hbuf.at[j]], add=True)
                return cc
            lax.fori_loop(0, 8, dscat, 0)
            return c
        lax.fori_loop(0, ngrp, grp, 0)

        def scale(j):
            def srow_g(g, cc):
                a16 = xbuf[j, pl.ds(g * LANES, LANES)]
                for ll in range(LANES):
                    i = g * LANES + ll
                    a = a16[ll]
                    for kk in range(d // LANES):
                        sl = pl.ds(kk * LANES, LANES)
                        rows[i, sl] = rows[i, sl] * a
                return cc
            lax.fori_loop(0, 8, srow_g, 0)

        def pipe(j, c):
            pltpu.async_copy(ent.at[tbuf.at[j]], rows, sema).wait()
            scale(j)
            pltpu.sync_copy(rows, ash.at[hbuf.at[j]], add=True)
            return c
        lax.fori_loop(0, rows_w, pipe, 0)
        plsc.subcore_barrier()

        pltpu.sync_copy(ash.at[pl.ds(sid * stripe, stripe)],
                        agg_out.at[cid, pl.ds(sid * stripe, stripe)])
        pltpu.sync_copy(dsh.at[pl.ds(sid * stripe, stripe)],
                        den_out.at[pl.ds(cid * npad + sid * stripe, stripe)])

    return k


def kernel(entity_emb, relation_emb, edge_index, edge_type, W):
    n, d = entity_emb.shape
    r = relation_emb.shape[0]
    e_real = edge_type.shape[0]
    rows_w = _rup(-(-e_real // (NW * 128)), 8)
    epad = NW * rows_w * 128
    npad = _rup(n, LANES * 128)
    blk = 1000 if n % 1000 == 0 else 8

    head = edge_index[0].astype(jnp.int32)
    tail = edge_index[1].astype(jnp.int32)
    etype = edge_type.astype(jnp.int32)
    pad = epad - e_real
    zpad = jnp.zeros((pad,), jnp.int32)
    head2 = jnp.concatenate([head, zpad]).reshape(NW * rows_w, 128)
    tail2 = jnp.concatenate([tail, zpad]).reshape(NW * rows_w, 128)
    type2 = jnp.concatenate([etype, zpad]).reshape(NW * rows_w, 128)

    proj = _make_proj(n, d, r, blk)
    shift = _make_shift(npad)
    norm = _make_norm(n, d, blk)
    sc_a = _make_sc_logits_den(rows_w, r, e_real, npad)
    sc_c = _make_sc_aggden(rows_w, e_real, npad, d)

    ent = entity_emb
    res = entity_emb
    for _ in range(N_HOPS):
        p1, p2 = proj(ent, W, relation_emb)
        p1f = p1.reshape(n * r)
        p2f = p2.reshape(n * r)
        e2, den1, pmax = sc_a(p1f, p2f, head2, tail2, type2)
        pm = pmax.reshape(LANES, 2, LANES)
        s2 = shift(pm[:, 0], pm[:, 1],
                   den1.reshape(2, npad // 128, 128)).reshape(npad)
        agg, den2 = sc_c(e2, s2, head2, tail2, ent)
        d0c = den2[:npad].reshape(npad, 1)
        d1c = den2[npad:].reshape(npad, 1)
        ent, res = norm(agg[0, :n], agg[1, :n], d0c[:n], d1c[:n], ent, res)
    return res


# speedup vs baseline: 9.3973x; 1.0989x over previous
import functools

import jax
import jax.numpy as jnp
from jax import lax
from jax.experimental import pallas as pl
from jax.experimental.pallas import tpu as pltpu
from jax.experimental.pallas import tpu_sc as plsc

NEG_SLOPE = 0.2
LAM = 0.5
N_HOPS = 2
NW = 32
LANES = 16


def _rup(x, m):
    return (x + m - 1) // m * m


def _proj_body(ent_ref, w_ref, rel_ref, p1_ref, p2_ref):
    d = ent_ref.shape[1]
    cdims = (((1,), (1,)), ((), ()))
    m1 = lax.dot_general(w_ref[0:d, :], rel_ref[...], cdims,
                         preferred_element_type=jnp.float32)
    m2 = lax.dot_general(w_ref[d:2 * d, :], rel_ref[...], cdims,
                         preferred_element_type=jnp.float32)
    e = ent_ref[...]
    p1_ref[...] = jnp.dot(e, m1, preferred_element_type=jnp.float32)
    p2_ref[...] = jnp.dot(e, m2, preferred_element_type=jnp.float32)


@functools.lru_cache(maxsize=None)
def _make_proj(n, d, r, blk):
    grid = n // blk
    return pl.pallas_call(
        _proj_body,
        grid=(grid,),
        in_specs=[
            pl.BlockSpec((blk, d), lambda i: (i, 0)),
            pl.BlockSpec((2 * d, d), lambda i: (0, 0)),
            pl.BlockSpec((r, d), lambda i: (0, 0)),
        ],
        out_specs=[
            pl.BlockSpec((blk, r), lambda i: (i, 0)),
            pl.BlockSpec((blk, r), lambda i: (i, 0)),
        ],
        out_shape=[
            jax.ShapeDtypeStruct((n, r), jnp.float32),
            jax.ShapeDtypeStruct((n, r), jnp.float32),
        ],
    )


def _shift_body(pm0_ref, pm1_ref, den_ref, s_ref):
    m0 = jnp.max(pm0_ref[...])
    m1 = jnp.max(pm1_ref[...])
    g = jnp.maximum(m0, m1)
    dt = jnp.exp(m0 - g) * den_ref[0] + jnp.exp(m1 - g) * den_ref[1]
    dts = jnp.where(dt > 0.0, dt, 1.0)
    s_ref[...] = jnp.where(dt > 0.0, g + jnp.log(dts), g - 88.0)


@functools.lru_cache(maxsize=None)
def _make_shift(npad):
    rows = npad // 128
    return pl.pallas_call(
        _shift_body,
        grid=(1,),
        in_specs=[
            pl.BlockSpec((LANES, LANES), lambda i: (0, 0)),
            pl.BlockSpec((LANES, LANES), lambda i: (0, 0)),
            pl.BlockSpec((2, rows, 128), lambda i: (0, 0, 0)),
        ],
        out_specs=pl.BlockSpec((rows, 128), lambda i: (0, 0)),
        out_shape=jax.ShapeDtypeStruct((rows, 128), jnp.float32),
    )


def _norm_body(a0_ref, a1_ref, d0_ref, d1_ref, ent_ref, res_ref,
               oent_ref, ores_ref):
    dt = d0_ref[...] + d1_ref[...]
    dts = jnp.where(dt > 0.0, dt, 1.0)
    a = (a0_ref[...] + a1_ref[...]) / dts + ent_ref[...]
    nrm = jnp.sqrt(jnp.sum(a * a, axis=1, keepdims=True))
    ent_new = a / jnp.maximum(nrm, 1e-12)
    oent_ref[...] = ent_new
    ores_ref[...] = LAM * res_ref[...] + ent_new


@functools.lru_cache(maxsize=None)
def _make_norm(n, d, blk):
    grid = n // blk
    return pl.pallas_call(
        _norm_body,
        grid=(grid,),
        in_specs=[
            pl.BlockSpec((blk, d), lambda i: (i, 0)),
            pl.BlockSpec((blk, d), lambda i: (i, 0)),
            pl.BlockSpec((blk, 1), lambda i: (i, 0)),
            pl.BlockSpec((blk, 1), lambda i: (i, 0)),
            pl.BlockSpec((blk, d), lambda i: (i, 0)),
            pl.BlockSpec((blk, d), lambda i: (i, 0)),
        ],
        out_specs=[
            pl.BlockSpec((blk, d), lambda i: (i, 0)),
            pl.BlockSpec((blk, d), lambda i: (i, 0)),
        ],
        out_shape=[
            jax.ShapeDtypeStruct((n, d), jnp.float32),
            jax.ShapeDtypeStruct((n, d), jnp.float32),
        ],
    )


def _mesh():
    return plsc.VectorSubcoreMesh(core_axis_name="c", subcore_axis_name="s")


def _wid():
    return lax.axis_index("s") * 2 + lax.axis_index("c")


def _butterfly_max(m):
    idx = lax.iota(jnp.int32, LANES)
    for s in (8, 4, 2, 1):
        m = jnp.maximum(m, m.at[idx ^ s].get(mode="promise_in_bounds"))
    return m[0]


@functools.lru_cache(maxsize=None)
def _make_sc_logits_den(rows_w, r, e_real, npad):
    nrows = NW * rows_w
    stripe = npad // LANES

    @functools.partial(
        pl.kernel, mesh=_mesh(),
        out_type=(jax.ShapeDtypeStruct((nrows, 128), jnp.float32),
                  jax.ShapeDtypeStruct((2 * npad,), jnp.float32),
                  jax.ShapeDtypeStruct((NW * LANES,), jnp.float32)),
        scratch_types=[
            pltpu.VMEM((rows_w, 128), jnp.int32),
            pltpu.VMEM((rows_w, 128), jnp.int32),
            pltpu.VMEM((rows_w, 128), jnp.int32),
            pltpu.VMEM((rows_w, 128), jnp.int32),
            pltpu.VMEM((rows_w, 128), jnp.int32),
            pltpu.VMEM((rows_w * 128,), jnp.float32),
            pltpu.VMEM((rows_w * 128,), jnp.float32),
            pltpu.VMEM((rows_w, 128), jnp.float32),
            pltpu.VMEM((rows_w, 128), jnp.float32),
            pltpu.VMEM((LANES,), jnp.float32),
            pltpu.VMEM((LANES * LANES,), jnp.float32),
            pltpu.VMEM((npad // LANES,), jnp.float32),
            pltpu.VMEM_SHARED((LANES * LANES,), jnp.float32),
            pltpu.VMEM_SHARED((npad,), jnp.float32),
            pltpu.SemaphoreType.DMA,
            pltpu.SemaphoreType.DMA,
        ])
    def k(p1f, p2f, head2, tail2, type2, e_out, den_out, pmax_out,
          hbuf, tbuf, ybuf, i1, i2, v1, v2, ebuf, xbuf, mbuf, pvb,
          zbuf, dshm, dsh, sem1, sem2):
        w = _wid()
        cid = lax.axis_index("c")
        sid = lax.axis_index("s")
        base = w * rows_w
        pltpu.sync_copy(head2.at[pl.ds(base, rows_w)], hbuf)
        pltpu.sync_copy(tail2.at[pl.ds(base, rows_w)], tbuf)
        pltpu.sync_copy(type2.at[pl.ds(base, rows_w)], ybuf)

        def idx_row(j, c):
            for kk in range(8):
                sl = pl.ds(kk * LANES, LANES)
                y = ybuf[j, sl]
                i1[j, sl] = hbuf[j, sl] * r + y
                i2[j, sl] = tbuf[j, sl] * r + y
            return c
        lax.fori_loop(0, rows_w, idx_row, 0)

        def fire(j, c):
            pltpu.async_copy(p1f.at[i1.at[j]],
                             v1.at[pl.ds(j * 128, 128)], sem1)
            pltpu.async_copy(p2f.at[i2.at[j]],
                             v2.at[pl.ds(j * 128, 128)], sem2)
            return c
        lax.fori_loop(0, rows_w, fire, 0)

        def zrow(q, c):
            zbuf[pl.ds(q * LANES, LANES)] = jnp.zeros((LANES,), jnp.float32)
            return c
        lax.fori_loop(0, stripe // LANES, zrow, 0)
        pltpu.sync_copy(zbuf, dsh.at[pl.ds(sid * stripe, stripe)])

        pltpu.make_async_copy(p1f.at[pl.ds(0, rows_w * 128)], v1, sem1).wait()
        pltpu.make_async_copy(p2f.at[pl.ds(0, rows_w * 128)], v2, sem2).wait()

        def e_row(j, m):
            for kk in range(8):
                sl = pl.ds(kk * LANES, LANES)
                fsl = pl.ds(j * 128 + kk * LANES, LANES)
                s = v1[fsl] + v2[fsl]
                ev = jnp.where(s >= 0.0, s, NEG_SLOPE * s)
                ebuf[j, sl] = ev
                m = jnp.maximum(m, ev)
            return m
        m = lax.fori_loop(0, rows_w, e_row,
                          jnp.full((LANES,), -3e38, jnp.float32))
        mbuf[...] = m
        pltpu.sync_copy(ebuf, e_out.at[pl.ds(base, rows_w)])
        pltpu.sync_copy(mbuf, pmax_out.at[pl.ds(w * LANES, LANES)])

        pltpu.sync_copy(mbuf, dshm.at[pl.ds(sid * LANES, LANES)])
        plsc.subcore_barrier()
        pltpu.sync_copy(dshm, pvb)

        def mrow(i, mm):
            return jnp.maximum(mm, pvb[pl.ds(i * LANES, LANES)])
        mc = _butterfly_max(lax.fori_loop(
            0, LANES, mrow, jnp.full((LANES,), -3e38, jnp.float32)))

        iota = lax.iota(jnp.int32, LANES)

        def x_row(j, c):
            gid0 = (base + j) * 128
            for kk in range(8):
                sl = pl.ds(kk * LANES, LANES)
                x = jnp.exp(ebuf[j, sl] - mc)
                gid = gid0 + kk * LANES + iota
                xbuf[j, sl] = jnp.where(gid < e_real, x, 0.0)
            return c
        lax.fori_loop(0, rows_w, x_row, 0)

        def scat(j, c):
            pltpu.sync_copy(xbuf.at[j], dsh.at[hbuf.at[j]], add=True)
            return c
        lax.fori_loop(0, rows_w, scat, 0)
        plsc.subcore_barrier()
        pltpu.sync_copy(dsh.at[pl.ds(sid * stripe, stripe)],
                        den_out.at[pl.ds(cid * npad + sid * stripe, stripe)])

    return k


@functools.lru_cache(maxsize=None)
def _make_sc_aggden(rows_w, e_real, npad, d):
    stripe = npad // LANES
    nsup = rows_w // 16

    @functools.partial(
        pl.kernel, mesh=_mesh(),
        out_type=(jax.ShapeDtypeStruct((2, npad, d), jnp.float32),
                  jax.ShapeDtypeStruct((2 * npad,), jnp.float32)),
        scratch_types=[
            pltpu.VMEM((16, 128), jnp.int32),
            pltpu.VMEM((16, 128), jnp.int32),
            pltpu.VMEM((16, 128), jnp.float32),
            pltpu.VMEM((16 * 128,), jnp.float32),
            pltpu.VMEM((128, d), jnp.float32),
            pltpu.VMEM((128, d), jnp.float32),
            pltpu.VMEM_SHARED((npad, d), jnp.float32),
            pltpu.VMEM_SHARED((npad,), jnp.float32),
            pltpu.SemaphoreType.DMA,
            pltpu.SemaphoreType.DMA,
            pltpu.SemaphoreType.DMA,
        ])
    def k(e2, s1d, head2, tail2, ent, agg_out, den_out,
          h16, t16, x16, sv, rows_a, rows_b, ash, dsh, semx, sema, semb):
        w = _wid()
        cid = lax.axis_index("c")
        sid = lax.axis_index("s")
        base = w * rows_w

        def zr(i, c):
            for kk in range(d // LANES):
                rows_a[i, pl.ds(kk * LANES, LANES)] = jnp.zeros(
                    (LANES,), jnp.float32)
            return c
        lax.fori_loop(0, 128, zr, 0)

        def zcopy(q, c):
            pltpu.sync_copy(rows_a, ash.at[pl.ds(sid * stripe + q * 128, 128)])
            return c
        lax.fori_loop(0, stripe // 128, zcopy, 0)

        def zs(q, c):
            sv[pl.ds(q * LANES, LANES)] = jnp.zeros((LANES,), jnp.float32)
            return c
        lax.fori_loop(0, (16 * 128) // LANES, zs, 0)
        pltpu.sync_copy(sv.at[pl.ds(0, stripe)],
                        dsh.at[pl.ds(sid * stripe, stripe)])
        plsc.subcore_barrier()

        iota = lax.iota(jnp.int32, LANES)

        def scale(rows, q):
            def srow_g(g, cc):
                a16 = x16[q, pl.ds(g * LANES, LANES)]
                for ll in range(LANES):
                    i = g * LANES + ll
                    a = a16[ll]
                    for kk in range(d // LANES):
                        sl = pl.ds(kk * LANES, LANES)
                        rows[i, sl] = rows[i, sl] * a
                return cc
            lax.fori_loop(0, 8, srow_g, 0)

        def sup(s, c):
            sbase = base + s * 16
            pltpu.sync_copy(head2.at[pl.ds(sbase, 16)], h16)
            pltpu.sync_copy(tail2.at[pl.ds(sbase, 16)], t16)
            pltpu.sync_copy(e2.at[pl.ds(sbase, 16)], x16)

            def fire_s(q, cc):
                pltpu.async_copy(s1d.at[h16.at[q]],
                                 sv.at[pl.ds(q * 128, 128)], semx)
                return cc
            lax.fori_loop(0, 16, fire_s, 0)
            pltpu.make_async_copy(s1d.at[pl.ds(0, 16 * 128)], sv,
                                  semx).wait()

            def x_row(q, cc):
                gid0 = (sbase + q) * 128
                for kk in range(8):
                    sl = pl.ds(kk * LANES, LANES)
                    fsl = pl.ds(q * 128 + kk * LANES, LANES)
                    x = jnp.exp(jnp.minimum(x16[q, sl] - sv[fsl], 1.0))
                    gid = gid0 + kk * LANES + iota
                    x16[q, sl] = jnp.where(gid < e_real, x, 0.0)
                return cc
            lax.fori_loop(0, 16, x_row, 0)

            def dscat(q, cc):
                pltpu.sync_copy(x16.at[q], dsh.at[h16.at[q]], add=True)
                return cc
            lax.fori_loop(0, 16, dscat, 0)

            pltpu.async_copy(ent.at[t16.at[0]], rows_a, sema)

            def pipe(q2, cc):
                q = 2 * q2
                pltpu.async_copy(ent.at[t16.at[q + 1]], rows_b, semb)
                pltpu.make_async_copy(ent.at[t16.at[0]], rows_a, sema).wait()
                scale(rows_a, q)
                pltpu.sync_copy(rows_a, ash.at[h16.at[q]], add=True)
                pltpu.async_copy(ent.at[t16.at[jnp.minimum(q + 2, 15)]],
                                 rows_a, sema)
                pltpu.make_async_copy(ent.at[t16.at[0]], rows_b, semb).wait()
                scale(rows_b, q + 1)
                pltpu.sync_copy(rows_b, ash.at[h16.at[q + 1]], add=True)
                return cc
            lax.fori_loop(0, 8, pipe, 0)
            pltpu.make_async_copy(ent.at[t16.at[0]], rows_a, sema).wait()
            return c
        lax.fori_loop(0, nsup, sup, 0)
        plsc.subcore_barrier()

        pltpu.sync_copy(ash.at[pl.ds(sid * stripe, stripe)],
                        agg_out.at[cid, pl.ds(sid * stripe, stripe)])
        pltpu.sync_copy(dsh.at[pl.ds(sid * stripe, stripe)],
                        den_out.at[pl.ds(cid * npad + sid * stripe, stripe)])

    return k


def kernel(entity_emb, relation_emb, edge_index, edge_type, W):
    n, d = entity_emb.shape
    r = relation_emb.shape[0]
    e_real = edge_type.shape[0]
    rows_w = _rup(-(-e_real // (NW * 128)), 8)
    epad = NW * rows_w * 128
    npad = _rup(n, LANES * 128)
    blk = 1000 if n % 1000 == 0 else 8

    head = edge_index[0].astype(jnp.int32)
    tail = edge_index[1].astype(jnp.int32)
    etype = edge_type.astype(jnp.int32)
    pad = epad - e_real
    zpad = jnp.zeros((pad,), jnp.int32)
    head2 = jnp.concatenate([head, zpad]).reshape(NW * rows_w, 128)
    tail2 = jnp.concatenate([tail, zpad]).reshape(NW * rows_w, 128)
    type2 = jnp.concatenate([etype, zpad]).reshape(NW * rows_w, 128)

    proj = _make_proj(n, d, r, blk)
    shift = _make_shift(npad)
    norm = _make_norm(n, d, blk)
    sc_a = _make_sc_logits_den(rows_w, r, e_real, npad)
    sc_c = _make_sc_aggden(rows_w, e_real, npad, d)

    ent = entity_emb
    res = entity_emb
    for _ in range(N_HOPS):
        p1, p2 = proj(ent, W, relation_emb)
        p1f = p1.reshape(n * r)
        p2f = p2.reshape(n * r)
        e2, den1, pmax = sc_a(p1f, p2f, head2, tail2, type2)
        pm = pmax.reshape(LANES, 2, LANES)
        s2 = shift(pm[:, 0], pm[:, 1],
                   den1.reshape(2, npad // 128, 128)).reshape(npad)
        agg, den2 = sc_c(e2, s2, head2, tail2, ent)
        d0c = den2[:npad].reshape(npad, 1)
        d1c = den2[npad:].reshape(npad, 1)
        ent, res = norm(agg[0, :n], agg[1, :n], d0c[:n], d1c[:n], ent, res)
    return res
